# baseline (speedup 1.0000x reference)
import functools
import math

import jax
import jax.numpy as jnp
from jax import lax
from jax.experimental import pallas as pl
from jax.experimental.pallas import tpu as pltpu
from jax.experimental.pallas import tpu_sc as plsc

DIM = 768
P = 8
B = 8
N = 1024
HW = 32
K = 4 * P

NC, NS = 2, 16
NW = NC * NS
QTOT = B * N
QPW = QTOT // NW
G = 32
NG = QPW // G


def _prep_idx_body(x_ref, refct_ref, wcat_ref, bcat_ref, wv_ref, wo_ref,
                   bv_ref, bo_ref, w_ref, idx_ref, w2_ref, b2_ref):
    @pl.when(pl.program_id(0) == 0)
    def _fold():
        w2_ref[...] = jnp.dot(wv_ref[...], wo_ref[...],
                              preferred_element_type=jnp.float32)
        b2_ref[...] = jnp.dot(bv_ref[...], wo_ref[...],
                              preferred_element_type=jnp.float32) + bo_ref[...]

    proj = jnp.dot(x_ref[0], wcat_ref[...],
                   preferred_element_type=jnp.float32) + bcat_ref[...]
    pt = proj.T
    g = (jnp.clip(refct_ref[0] + pt[0:2 * P], -1.0, 1.0) + 1.0) * 0.5 * (HW - 1)
    f = jnp.floor(g)
    t = g - f
    f1 = jnp.minimum(f + 1.0, float(HW - 1))
    logits = pt[2 * P:3 * P]
    m = jnp.max(logits, axis=0, keepdims=True)
    e = jnp.exp(logits - m)
    aw = e / jnp.sum(e, axis=0, keepdims=True)
    tx = t[0:P]
    ty = t[P:2 * P]
    x0 = f[0:P]
    y0 = f[P:2 * P]
    x1 = f1[0:P]
    y1 = f1[P:2 * P]
    wx = jnp.concatenate([1.0 - tx, tx, 1.0 - tx, tx], axis=0)
    wy = jnp.concatenate([1.0 - ty, 1.0 - ty, ty, ty], axis=0)
    aw4 = jnp.concatenate([aw, aw, aw, aw], axis=0)
    xs = jnp.concatenate([x0, x1, x0, x1], axis=0)
    ys = jnp.concatenate([y0, y0, y1, y1], axis=0)
    w_ref[...] = aw4 * wx * wy
    idx_ref[...] = (ys * float(HW) + xs).astype(jnp.int32)


def _sc_scatter_body(w_hbm, i_hbm, s_hbm, w_v, i_v, buf0, buf1, sem0, sem1):
    wid = lax.axis_index("s") * NC + lax.axis_index("c")
    wpb = N // QPW
    bat = wid // wpb
    n0 = (wid % wpb) * QPW
    bufs = (buf0, buf1)
    sems = (sem0, sem1)
    zero16 = jnp.zeros((16,), jnp.float32)

    pltpu.sync_copy(w_hbm.at[pl.ds(bat * K, K), pl.ds(n0, QPW)], w_v)
    pltpu.sync_copy(i_hbm.at[pl.ds(bat * K, K), pl.ds(n0, QPW)], i_v)

    def _zero(k, _):
        for u in range(16):
            off = (k * 256 + u * 16) // N
            col = (k * 256 + u * 16) % N
            buf0[off, pl.ds(col, 16)] = zero16
            buf1[off, pl.ds(col, 16)] = zero16
        return 0

    lax.fori_loop(0, G * N // 256, _zero, 0)

    lane = lax.iota(jnp.int32, 16)
    copies = [None, None]
    for grp in range(NG):
        slot = grp % 2
        buf = bufs[slot]
        if copies[slot] is not None:
            copies[slot].wait()
            for sub in range(G // 16):
                rows = lane + sub * 16
                q0 = (grp - 2) * G + sub * 16
                for pc in range(K):
                    iv = i_v[pc, pl.ds(q0, 16)]
                    plsc.store_scatter(buf, [rows, iv], zero16)
        for sub in range(G // 16):
            rows = lane + sub * 16
            q0 = grp * G + sub * 16
            for pc in range(K):
                wv = w_v[pc, pl.ds(q0, 16)]
                iv = i_v[pc, pl.ds(q0, 16)]
                plsc.addupdate_scatter(buf, [rows, iv], wv)
        copies[slot] = pltpu.async_copy(
            buf, s_hbm.at[bat, pl.ds(n0 + grp * G, G), :], sems[slot])
    copies[0].wait()
    copies[1].wait()


def _apply_body(sa_ref, sb_ref, x_ref, w2_ref, b2_ref, out_ref):
    bf = jnp.bfloat16
    xw = jnp.dot(x_ref[0].astype(bf), w2_ref[...].astype(bf),
                 preferred_element_type=jnp.float32).astype(bf)
    out_ref[0, 0:N // 2] = jnp.dot(sa_ref[0].astype(bf), xw,
                                   preferred_element_type=jnp.float32) + b2_ref[...]
    out_ref[0, N // 2:N] = jnp.dot(sb_ref[0].astype(bf), xw,
                                   preferred_element_type=jnp.float32) + b2_ref[...]


def kernel(x, ref_points, Wv, bv, Woff, boff, Waw, baw, Wo, bo):
    f32 = jnp.float32
    Wcat = jnp.concatenate([Woff[:, 0::2], Woff[:, 1::2], Waw], axis=1)
    bcat = jnp.concatenate([boff[0::2], boff[1::2], baw])[None, :]
    refct = jnp.concatenate(
        [jnp.tile(ref_points[:, None, :, 0], (1, P, 1)),
         jnp.tile(ref_points[:, None, :, 1], (1, P, 1))], axis=1)

    w, idx, W2, b2 = pl.pallas_call(
        _prep_idx_body,
        grid=(B,),
        in_specs=[
            pl.BlockSpec((1, N, DIM), lambda b: (b, 0, 0)),
            pl.BlockSpec((1, 2 * P, N), lambda b: (b, 0, 0)),
            pl.BlockSpec((DIM, 3 * P), lambda b: (0, 0)),
            pl.BlockSpec((1, 3 * P), lambda b: (0, 0)),
            pl.BlockSpec((DIM, DIM), lambda b: (0, 0)),
            pl.BlockSpec((DIM, DIM), lambda b: (0, 0)),
            pl.BlockSpec((1, DIM), lambda b: (0, 0)),
            pl.BlockSpec((1, DIM), lambda b: (0, 0)),
        ],
        out_specs=[
            pl.BlockSpec((K, N), lambda b: (b, 0)),
            pl.BlockSpec((K, N), lambda b: (b, 0)),
            pl.BlockSpec((DIM, DIM), lambda b: (0, 0)),
            pl.BlockSpec((1, DIM), lambda b: (0, 0)),
        ],
        out_shape=[
            jax.ShapeDtypeStruct((B * K, N), f32),
            jax.ShapeDtypeStruct((B * K, N), jnp.int32),
            jax.ShapeDtypeStruct((DIM, DIM), f32),
            jax.ShapeDtypeStruct((1, DIM), f32),
        ],
        compiler_params=pltpu.CompilerParams(
            dimension_semantics=("arbitrary",)),
    )(x, refct, Wcat, bcat, Wv, Wo, bv[None, :], bo[None, :])

    sc_scatter = pl.kernel(
        _sc_scatter_body,
        out_type=jax.ShapeDtypeStruct((B, N, N), f32),
        mesh=plsc.VectorSubcoreMesh(core_axis_name="c", subcore_axis_name="s"),
        scratch_types=[
            pltpu.VMEM((K, QPW), f32),
            pltpu.VMEM((K, QPW), jnp.int32),
            pltpu.VMEM((G, N), f32),
            pltpu.VMEM((G, N), f32),
            pltpu.SemaphoreType.DMA,
            pltpu.SemaphoreType.DMA,
        ],
        compiler_params=pltpu.CompilerParams(needs_layout_passes=False),
    )
    s = sc_scatter(w, idx)

    out = pl.pallas_call(
        _apply_body,
        grid=(B,),
        in_specs=[
            pl.BlockSpec((1, N // 2, N), lambda b: (b, 0, 0)),
            pl.BlockSpec((1, N // 2, N), lambda b: (b, 1, 0)),
            pl.BlockSpec((1, N, DIM), lambda b: (b, 0, 0)),
            pl.BlockSpec((DIM, DIM), lambda b: (0, 0)),
            pl.BlockSpec((1, DIM), lambda b: (0, 0)),
        ],
        out_specs=pl.BlockSpec((1, N, DIM), lambda b: (b, 0, 0)),
        out_shape=jax.ShapeDtypeStruct((B, N, DIM), f32),
        compiler_params=pltpu.CompilerParams(
            dimension_semantics=("parallel",)),
    )(s, s, x, W2, b2)
    return out


# speedup vs baseline: 1.0056x; 1.0056x over previous
import jax
import jax.numpy as jnp
from jax import lax
from jax.experimental import pallas as pl
from jax.experimental.pallas import tpu as pltpu
from jax.experimental.pallas import tpu_sc as plsc

DIM = 768
P = 8
B = 8
N = 1024
HW = 32
K = 4 * P

NC, NS = 2, 16
NW = NC * NS
QTOT = B * N
QPW = QTOT // NW
G = 32
NG = QPW // G


def _prep_idx_body(x_ref, refct_ref, wcat_ref, bcat_ref, wv_ref, wo_ref,
                   bv_ref, bo_ref, w_ref, idx_ref, w2_ref, b2_ref):
    @pl.when(pl.program_id(0) == 0)
    def _fold():
        w2_ref[...] = jnp.dot(wv_ref[...], wo_ref[...],
                              preferred_element_type=jnp.float32)
        b2_ref[...] = jnp.dot(bv_ref[...], wo_ref[...],
                              preferred_element_type=jnp.float32) + bo_ref[...]

    proj = jnp.dot(x_ref[0], wcat_ref[...],
                   preferred_element_type=jnp.float32) + bcat_ref[...]
    pt = proj.T
    g = (jnp.clip(refct_ref[0] + pt[0:2 * P], -1.0, 1.0) + 1.0) * 0.5 * (HW - 1)
    f = jnp.floor(g)
    t = g - f
    f1 = jnp.minimum(f + 1.0, float(HW - 1))
    logits = pt[2 * P:3 * P]
    m = jnp.max(logits, axis=0, keepdims=True)
    e = jnp.exp(logits - m)
    aw = e / jnp.sum(e, axis=0, keepdims=True)
    tx = t[0:P]
    ty = t[P:2 * P]
    x0 = f[0:P]
    y0 = f[P:2 * P]
    x1 = f1[0:P]
    y1 = f1[P:2 * P]
    wx = jnp.concatenate([1.0 - tx, tx, 1.0 - tx, tx], axis=0)
    wy = jnp.concatenate([1.0 - ty, 1.0 - ty, ty, ty], axis=0)
    aw4 = jnp.concatenate([aw, aw, aw, aw], axis=0)
    xs = jnp.concatenate([x0, x1, x0, x1], axis=0)
    ys = jnp.concatenate([y0, y0, y1, y1], axis=0)
    w_ref[...] = aw4 * wx * wy
    idx_ref[...] = (ys * float(HW) + xs).astype(jnp.int32)


def _sc_scatter_body(w_hbm, i_hbm, s_hbm, w_v, i_v, buf0, buf1, sem0, sem1):
    wid = lax.axis_index("s") * NC + lax.axis_index("c")
    wpb = N // QPW
    bat = wid // wpb
    n0 = (wid % wpb) * QPW
    bufs = (buf0, buf1)
    sems = (sem0, sem1)
    zero16 = jnp.zeros((16,), jnp.float32)

    pltpu.sync_copy(w_hbm.at[pl.ds(bat * K, K), pl.ds(n0, QPW)], w_v)
    pltpu.sync_copy(i_hbm.at[pl.ds(bat * K, K), pl.ds(n0, QPW)], i_v)

    def _zero(k, _):
        for u in range(16):
            off = (k * 256 + u * 16) // N
            col = (k * 256 + u * 16) % N
            buf0[off, pl.ds(col, 16)] = zero16
            buf1[off, pl.ds(col, 16)] = zero16
        return 0

    lax.fori_loop(0, G * N // 256, _zero, 0)

    lane = lax.iota(jnp.int32, 16)
    copies = [None, None]
    for grp in range(NG):
        slot = grp % 2
        buf = bufs[slot]
        if copies[slot] is not None:
            copies[slot].wait()
            for sub in range(G // 16):
                rows = lane + sub * 16
                q0 = (grp - 2) * G + sub * 16
                for pc in range(K):
                    iv = i_v[pc, pl.ds(q0, 16)]
                    plsc.store_scatter(buf, [rows, iv], zero16)
        for sub in range(G // 16):
            rows = lane + sub * 16
            q0 = grp * G + sub * 16
            for pc in range(K):
                wv = w_v[pc, pl.ds(q0, 16)]
                iv = i_v[pc, pl.ds(q0, 16)]
                plsc.addupdate_scatter(buf, [rows, iv], wv)
        copies[slot] = pltpu.async_copy(
            buf, s_hbm.at[bat, pl.ds(n0 + grp * G, G), :], sems[slot])
    copies[0].wait()
    copies[1].wait()


def _apply_body(s_ref, x_ref, w2_ref, b2_ref, out_ref):
    bf = jnp.bfloat16
    xw = jnp.dot(x_ref[0].astype(bf), w2_ref[...].astype(bf),
                 preferred_element_type=jnp.float32)
    out_ref[0] = jnp.dot(s_ref[0].astype(bf), xw.astype(bf),
                         preferred_element_type=jnp.float32) + b2_ref[...]


def kernel(x, ref_points, Wv, bv, Woff, boff, Waw, baw, Wo, bo):
    f32 = jnp.float32
    Wcat = jnp.concatenate([Woff[:, 0::2], Woff[:, 1::2], Waw], axis=1)
    bcat = jnp.concatenate([boff[0::2], boff[1::2], baw])[None, :]
    refct = jnp.concatenate(
        [jnp.tile(ref_points[:, None, :, 0], (1, P, 1)),
         jnp.tile(ref_points[:, None, :, 1], (1, P, 1))], axis=1)

    w, idx, W2, b2 = pl.pallas_call(
        _prep_idx_body,
        grid=(B,),
        in_specs=[
            pl.BlockSpec((1, N, DIM), lambda b: (b, 0, 0)),
            pl.BlockSpec((1, 2 * P, N), lambda b: (b, 0, 0)),
            pl.BlockSpec((DIM, 3 * P), lambda b: (0, 0)),
            pl.BlockSpec((1, 3 * P), lambda b: (0, 0)),
            pl.BlockSpec((DIM, DIM), lambda b: (0, 0)),
            pl.BlockSpec((DIM, DIM), lambda b: (0, 0)),
            pl.BlockSpec((1, DIM), lambda b: (0, 0)),
            pl.BlockSpec((1, DIM), lambda b: (0, 0)),
        ],
        out_specs=[
            pl.BlockSpec((K, N), lambda b: (b, 0)),
            pl.BlockSpec((K, N), lambda b: (b, 0)),
            pl.BlockSpec((DIM, DIM), lambda b: (0, 0)),
            pl.BlockSpec((1, DIM), lambda b: (0, 0)),
        ],
        out_shape=[
            jax.ShapeDtypeStruct((B * K, N), f32),
            jax.ShapeDtypeStruct((B * K, N), jnp.int32),
            jax.ShapeDtypeStruct((DIM, DIM), f32),
            jax.ShapeDtypeStruct((1, DIM), f32),
        ],
        compiler_params=pltpu.CompilerParams(
            dimension_semantics=("arbitrary",)),
    )(x, refct, Wcat, bcat, Wv, Wo, bv[None, :], bo[None, :])

    sc_scatter = pl.kernel(
        _sc_scatter_body,
        out_type=jax.ShapeDtypeStruct((B, N, N), f32),
        mesh=plsc.VectorSubcoreMesh(core_axis_name="c", subcore_axis_name="s"),
        scratch_types=[
            pltpu.VMEM((K, QPW), f32),
            pltpu.VMEM((K, QPW), jnp.int32),
            pltpu.VMEM((G, N), f32),
            pltpu.VMEM((G, N), f32),
            pltpu.SemaphoreType.DMA,
            pltpu.SemaphoreType.DMA,
        ],
        compiler_params=pltpu.CompilerParams(needs_layout_passes=False),
    )
    s = sc_scatter(w, idx)

    out = pl.pallas_call(
        _apply_body,
        grid=(B,),
        in_specs=[
            pl.BlockSpec((1, N, N), lambda b: (b, 0, 0)),
            pl.BlockSpec((1, N, DIM), lambda b: (b, 0, 0)),
            pl.BlockSpec((DIM, DIM), lambda b: (0, 0)),
            pl.BlockSpec((1, DIM), lambda b: (0, 0)),
        ],
        out_specs=pl.BlockSpec((1, N, DIM), lambda b: (b, 0, 0)),
        out_shape=jax.ShapeDtypeStruct((B, N, DIM), f32),
        compiler_params=pltpu.CompilerParams(
            dimension_semantics=("parallel",)),
    )(s, x, W2, b2)
    return out
